# baseline (speedup 1.0000x reference)
import functools

import jax
import jax.numpy as jnp
from jax import lax
from jax.experimental import pallas as pl
from jax.experimental.pallas import tpu as pltpu
from jax.experimental.pallas import tpu_sc as plsc

Z = 128
R = 64
NCOMBO = 4


def _softplus(x):
    return jnp.maximum(x, 0.0) + jnp.log1p(jnp.exp(-jnp.abs(x)))


def _table_kernel(mu_c_ref, lr_c_ref, d_c_ref, mu_s_ref, lr_s_ref, d_s_ref,
                  cov_t_ref, mu_t_ref):
    row = jax.lax.broadcasted_iota(jnp.int32, (Z, Z), 0)
    col = jax.lax.broadcasted_iota(jnp.int32, (Z, Z), 1)
    diag_mask = (row == col).astype(jnp.float32)
    zeros_blk = jnp.zeros((Z, Z), dtype=jnp.float32)
    for combo in range(NCOMBO):
        e = combo % 2
        lrc = lr_c_ref[e]
        cc = jax.lax.dot_general(lrc, lrc, (((1,), (1,)), ((), ())),
                                 preferred_element_type=jnp.float32)
        dc = _softplus(d_c_ref[e]) + 1e-6
        cc = cc + diag_mask * dc[None, :]
        lrs = lr_s_ref[combo]
        cs = jax.lax.dot_general(lrs, lrs, (((1,), (1,)), ((), ())),
                                 preferred_element_type=jnp.float32)
        ds = _softplus(d_s_ref[combo]) + 1e-6
        cs = cs + diag_mask * ds[None, :]
        cov_t_ref[combo, 0:Z, 0:Z] = cc
        cov_t_ref[combo, 0:Z, Z:2 * Z] = zeros_blk
        cov_t_ref[combo, Z:2 * Z, 0:Z] = zeros_blk
        cov_t_ref[combo, Z:2 * Z, Z:2 * Z] = cs
        mu_t_ref[combo, 0:Z] = mu_c_ref[e, :]
        mu_t_ref[combo, Z:2 * Z] = mu_s_ref[combo, :]


def _build_tables(mu_causal, low_rank_causal, diag_causal,
                  mu_spurious, low_rank_spurious, diag_spurious):
    mu_s = mu_spurious.reshape(NCOMBO, Z)
    lr_s = low_rank_spurious.reshape(NCOMBO, Z, R)
    d_s = diag_spurious.reshape(NCOMBO, Z)
    return pl.pallas_call(
        _table_kernel,
        out_shape=(
            jax.ShapeDtypeStruct((NCOMBO, 2 * Z, 2 * Z), jnp.float32),
            jax.ShapeDtypeStruct((NCOMBO, 2 * Z), jnp.float32),
        ),
    )(mu_causal, low_rank_causal, diag_causal, mu_s, lr_s, d_s)


W = 32


def _cov_kernel(combo_ref, cov_t_ref, cov_out_ref, sems):
    b = cov_out_ref.shape[0]

    def body(j, carry):
        @pl.when(j >= W)
        def _():
            pltpu.make_async_copy(cov_t_ref.at[0], cov_out_ref.at[j - W],
                                  sems.at[j % W]).wait()
        c = combo_ref[j]
        pltpu.make_async_copy(cov_t_ref.at[c], cov_out_ref.at[j],
                              sems.at[j % W]).start()
        return carry

    jax.lax.fori_loop(0, b, body, 0)

    def tail(j, carry):
        pltpu.make_async_copy(cov_t_ref.at[0], cov_out_ref.at[j],
                              sems.at[j % W]).wait()
        return carry

    jax.lax.fori_loop(b - W, b, tail, 0)


def _assemble_cov(combo, cov_t, b):
    return pl.pallas_call(
        _cov_kernel,
        grid_spec=pltpu.PrefetchScalarGridSpec(
            num_scalar_prefetch=1,
            grid=(1,),
            in_specs=[
                pl.BlockSpec((NCOMBO, 2 * Z, 2 * Z), lambda i, c: (0, 0, 0)),
            ],
            out_specs=pl.BlockSpec(memory_space=pl.ANY),
            scratch_shapes=[pltpu.SemaphoreType.DMA((W,))],
        ),
        out_shape=jax.ShapeDtypeStruct((b, 2 * Z, 2 * Z), jnp.float32),
    )(combo, cov_t)


def _sc_mu_gather(b):
    nc, ns = 1, 4
    nw = nc * ns
    bpw = b // nw
    mesh = plsc.VectorSubcoreMesh(core_axis_name="c", subcore_axis_name="s",
                                  num_cores=nc, num_subcores=ns)

    @functools.partial(
        pl.kernel,
        mesh=mesh,
        out_type=jax.ShapeDtypeStruct((b, 2 * Z), jnp.float32),
        scratch_types=[
            pltpu.VMEM((bpw,), jnp.int32),
            pltpu.VMEM((bpw, 2 * Z), jnp.float32),
            pltpu.SemaphoreType.DMA,
        ],
    )
    def k(mu_t, combo, mu_out, combo_v, rows_v, sem):
        wid = lax.axis_index("s") * nc + lax.axis_index("c")
        base = wid * bpw
        pltpu.sync_copy(combo.at[pl.ds(base, bpw)], combo_v)
        pltpu.async_copy(mu_t.at[combo_v], rows_v, sem).wait()
        pltpu.sync_copy(rows_v, mu_out.at[pl.ds(base, bpw)])

    return k


def kernel(y, e, mu_causal, low_rank_causal, diag_causal,
           mu_spurious, low_rank_spurious, diag_spurious):
    combo = (y.astype(jnp.int32) * 2 + e.astype(jnp.int32))
    b = y.shape[0]
    cov_t, mu_t = _build_tables(mu_causal, low_rank_causal, diag_causal,
                                mu_spurious, low_rank_spurious, diag_spurious)
    mu = _sc_mu_gather(b)(mu_t, combo)
    cov = _assemble_cov(combo, cov_t, b)
    return (mu, cov)


# speedup vs baseline: 1.0119x; 1.0119x over previous
import functools

import jax
import jax.numpy as jnp
from jax import lax
from jax.experimental import pallas as pl
from jax.experimental.pallas import tpu as pltpu
from jax.experimental.pallas import tpu_sc as plsc

Z = 128
R = 64
NCOMBO = 4


def _softplus(x):
    return jnp.maximum(x, 0.0) + jnp.log1p(jnp.exp(-jnp.abs(x)))


def _table_kernel(mu_c_ref, lr_c_ref, d_c_ref, mu_s_ref, lr_s_ref, d_s_ref,
                  cov_t_ref, mu_t_ref):
    row = jax.lax.broadcasted_iota(jnp.int32, (Z, Z), 0)
    col = jax.lax.broadcasted_iota(jnp.int32, (Z, Z), 1)
    diag_mask = (row == col).astype(jnp.float32)
    zeros_blk = jnp.zeros((Z, Z), dtype=jnp.float32)
    for combo in range(NCOMBO):
        e = combo % 2
        lrc = lr_c_ref[e]
        cc = jax.lax.dot_general(lrc, lrc, (((1,), (1,)), ((), ())),
                                 preferred_element_type=jnp.float32)
        dc = _softplus(d_c_ref[e]) + 1e-6
        cc = cc + diag_mask * dc[None, :]
        lrs = lr_s_ref[combo]
        cs = jax.lax.dot_general(lrs, lrs, (((1,), (1,)), ((), ())),
                                 preferred_element_type=jnp.float32)
        ds = _softplus(d_s_ref[combo]) + 1e-6
        cs = cs + diag_mask * ds[None, :]
        cov_t_ref[combo, 0:Z, 0:Z] = cc
        cov_t_ref[combo, 0:Z, Z:2 * Z] = zeros_blk
        cov_t_ref[combo, Z:2 * Z, 0:Z] = zeros_blk
        cov_t_ref[combo, Z:2 * Z, Z:2 * Z] = cs
        mu_t_ref[combo, 0:Z] = mu_c_ref[e, :]
        mu_t_ref[combo, Z:2 * Z] = mu_s_ref[combo, :]


def _build_tables(mu_causal, low_rank_causal, diag_causal,
                  mu_spurious, low_rank_spurious, diag_spurious):
    mu_s = mu_spurious.reshape(NCOMBO, Z)
    lr_s = low_rank_spurious.reshape(NCOMBO, Z, R)
    d_s = diag_spurious.reshape(NCOMBO, Z)
    return pl.pallas_call(
        _table_kernel,
        out_shape=(
            jax.ShapeDtypeStruct((NCOMBO, 2 * Z, 2 * Z), jnp.float32),
            jax.ShapeDtypeStruct((NCOMBO, 2 * Z), jnp.float32),
        ),
    )(mu_causal, low_rank_causal, diag_causal, mu_s, lr_s, d_s)


W = 32


def _cov_kernel(combo_ref, cov_t_ref, cov_out_ref, sems):
    b = cov_out_ref.shape[0]

    def body(j, carry):
        @pl.when(j >= W)
        def _():
            pltpu.make_async_copy(cov_t_ref.at[0], cov_out_ref.at[j - W],
                                  sems.at[j % W]).wait()
        c = combo_ref[j]
        pltpu.make_async_copy(cov_t_ref.at[c], cov_out_ref.at[j],
                              sems.at[j % W]).start()
        return carry

    jax.lax.fori_loop(0, b, body, 0)

    def tail(j, carry):
        pltpu.make_async_copy(cov_t_ref.at[0], cov_out_ref.at[j],
                              sems.at[j % W]).wait()
        return carry

    jax.lax.fori_loop(b - W, b, tail, 0)


def _assemble_cov(combo, cov_t, b):
    return pl.pallas_call(
        _cov_kernel,
        grid_spec=pltpu.PrefetchScalarGridSpec(
            num_scalar_prefetch=1,
            grid=(1,),
            in_specs=[
                pl.BlockSpec((NCOMBO, 2 * Z, 2 * Z), lambda i, c: (0, 0, 0)),
            ],
            out_specs=pl.BlockSpec(memory_space=pl.ANY),
            scratch_shapes=[pltpu.SemaphoreType.DMA((W,))],
        ),
        out_shape=jax.ShapeDtypeStruct((b, 2 * Z, 2 * Z), jnp.float32),
    )(combo, cov_t)


def _sc_mu_gather(b):
    info = plsc.get_sparse_core_info()
    nc, ns = 1, info.num_subcores
    nw = nc * ns
    bpw = b // nw
    mesh = plsc.VectorSubcoreMesh(core_axis_name="c", subcore_axis_name="s",
                                  num_cores=nc)

    @functools.partial(
        pl.kernel,
        mesh=mesh,
        out_type=jax.ShapeDtypeStruct((b, 2 * Z), jnp.float32),
        scratch_types=[
            pltpu.VMEM((bpw,), jnp.int32),
            pltpu.VMEM((bpw, 2 * Z), jnp.float32),
            pltpu.SemaphoreType.DMA,
        ],
    )
    def k(mu_t, combo, mu_out, combo_v, rows_v, sem):
        wid = lax.axis_index("s") * nc + lax.axis_index("c")
        base = wid * bpw
        pltpu.sync_copy(combo.at[pl.ds(base, bpw)], combo_v)
        pltpu.async_copy(mu_t.at[combo_v], rows_v, sem).wait()
        pltpu.sync_copy(rows_v, mu_out.at[pl.ds(base, bpw)])

    return k


def kernel(y, e, mu_causal, low_rank_causal, diag_causal,
           mu_spurious, low_rank_spurious, diag_spurious):
    combo = (y.astype(jnp.int32) * 2 + e.astype(jnp.int32))
    b = y.shape[0]
    cov_t, mu_t = _build_tables(mu_causal, low_rank_causal, diag_causal,
                                mu_spurious, low_rank_spurious, diag_spurious)
    mu = _sc_mu_gather(b)(mu_t, combo)
    cov = _assemble_cov(combo, cov_t, b)
    return (mu, cov)


# speedup vs baseline: 1.1428x; 1.1294x over previous
import functools

import jax
import jax.numpy as jnp
from jax import lax
from jax.experimental import pallas as pl
from jax.experimental.pallas import tpu as pltpu
from jax.experimental.pallas import tpu_sc as plsc

Z = 128
R = 64
NCOMBO = 4


def _softplus(x):
    return jnp.maximum(x, 0.0) + jnp.log1p(jnp.exp(-jnp.abs(x)))


def _table_kernel(mu_c_ref, lr_c_ref, d_c_ref, mu_s_ref, lr_s_ref, d_s_ref,
                  cov_t_ref, mu_t_ref):
    row = jax.lax.broadcasted_iota(jnp.int32, (Z, Z), 0)
    col = jax.lax.broadcasted_iota(jnp.int32, (Z, Z), 1)
    diag_mask = (row == col).astype(jnp.float32)
    zeros_blk = jnp.zeros((Z, Z), dtype=jnp.float32)
    for combo in range(NCOMBO):
        e = combo % 2
        lrc = lr_c_ref[e]
        cc = jax.lax.dot_general(lrc, lrc, (((1,), (1,)), ((), ())),
                                 preferred_element_type=jnp.float32)
        dc = _softplus(d_c_ref[e]) + 1e-6
        cc = cc + diag_mask * dc[None, :]
        lrs = lr_s_ref[combo]
        cs = jax.lax.dot_general(lrs, lrs, (((1,), (1,)), ((), ())),
                                 preferred_element_type=jnp.float32)
        ds = _softplus(d_s_ref[combo]) + 1e-6
        cs = cs + diag_mask * ds[None, :]
        cov_t_ref[combo, 0:Z, 0:Z] = cc
        cov_t_ref[combo, 0:Z, Z:2 * Z] = zeros_blk
        cov_t_ref[combo, Z:2 * Z, 0:Z] = zeros_blk
        cov_t_ref[combo, Z:2 * Z, Z:2 * Z] = cs
        mu_t_ref[combo, 0:Z] = mu_c_ref[e, :]
        mu_t_ref[combo, Z:2 * Z] = mu_s_ref[combo, :]


def _build_tables(mu_causal, low_rank_causal, diag_causal,
                  mu_spurious, low_rank_spurious, diag_spurious):
    mu_s = mu_spurious.reshape(NCOMBO, Z)
    lr_s = low_rank_spurious.reshape(NCOMBO, Z, R)
    d_s = diag_spurious.reshape(NCOMBO, Z)
    return pl.pallas_call(
        _table_kernel,
        out_shape=(
            jax.ShapeDtypeStruct((NCOMBO, 2 * Z, 2 * Z), jnp.float32),
            jax.ShapeDtypeStruct((NCOMBO, 2 * Z), jnp.float32),
        ),
    )(mu_causal, low_rank_causal, diag_causal, mu_s, lr_s, d_s)


W = 32


def _cov_kernel(combo_ref, cov_t_ref, cov_out_ref, sems):
    b = cov_out_ref.shape[0]

    def body(j, carry):
        @pl.when(j >= W)
        def _():
            pltpu.make_async_copy(cov_t_ref.at[0], cov_out_ref.at[j - W],
                                  sems.at[j % W]).wait()
        c = combo_ref[j]
        pltpu.make_async_copy(cov_t_ref.at[c], cov_out_ref.at[j],
                              sems.at[j % W]).start()
        return carry

    jax.lax.fori_loop(0, b, body, 0)

    def tail(j, carry):
        pltpu.make_async_copy(cov_t_ref.at[0], cov_out_ref.at[j],
                              sems.at[j % W]).wait()
        return carry

    jax.lax.fori_loop(b - W, b, tail, 0)


def _assemble_cov(combo, cov_t, b):
    return pl.pallas_call(
        _cov_kernel,
        grid_spec=pltpu.PrefetchScalarGridSpec(
            num_scalar_prefetch=1,
            grid=(1,),
            in_specs=[
                pl.BlockSpec((NCOMBO, 2 * Z, 2 * Z), lambda i, c: (0, 0, 0)),
            ],
            out_specs=pl.BlockSpec(memory_space=pl.ANY),
            scratch_shapes=[pltpu.SemaphoreType.DMA((W,))],
        ),
        out_shape=jax.ShapeDtypeStruct((b, 2 * Z, 2 * Z), jnp.float32),
    )(combo, cov_t)


def _sc_mu_gather(b):
    info = plsc.get_sparse_core_info()
    nc, ns = info.num_cores, info.num_subcores
    nw = nc * ns
    bpw = b // nw
    mesh = plsc.VectorSubcoreMesh(core_axis_name="c", subcore_axis_name="s")

    @functools.partial(
        pl.kernel,
        mesh=mesh,
        out_type=jax.ShapeDtypeStruct((b, 2 * Z), jnp.float32),
        scratch_types=[
            pltpu.VMEM((bpw,), jnp.int32),
            pltpu.VMEM((NCOMBO, 2 * Z), jnp.float32),
            pltpu.VMEM((bpw, 2 * Z), jnp.float32),
        ],
    )
    def k(mu_t, combo, mu_out, combo_v, mu_t_v, rows_v):
        wid = lax.axis_index("s") * nc + lax.axis_index("c")
        base = wid * bpw
        pltpu.sync_copy(combo.at[pl.ds(base, bpw)], combo_v)
        pltpu.sync_copy(mu_t, mu_t_v)
        cvecs = [combo_v[pl.ds(16 * m, 16)] for m in range(bpw // 16)]
        cs = [cvecs[m][i] for m in range(bpw // 16) for i in range(16)]
        for p in range(2 * Z // 16):
            t0 = mu_t_v[0, pl.ds(16 * p, 16)]
            t1 = mu_t_v[1, pl.ds(16 * p, 16)]
            t2 = mu_t_v[2, pl.ds(16 * p, 16)]
            t3 = mu_t_v[3, pl.ds(16 * p, 16)]
            for j in range(bpw):
                c = cs[j]
                r = jnp.where(c == 0, t0,
                              jnp.where(c == 1, t1,
                                        jnp.where(c == 2, t2, t3)))
                rows_v[j, pl.ds(16 * p, 16)] = r
        pltpu.sync_copy(rows_v, mu_out.at[pl.ds(base, bpw)])

    return k


def kernel(y, e, mu_causal, low_rank_causal, diag_causal,
           mu_spurious, low_rank_spurious, diag_spurious):
    combo = (y.astype(jnp.int32) * 2 + e.astype(jnp.int32))
    b = y.shape[0]
    cov_t, mu_t = _build_tables(mu_causal, low_rank_causal, diag_causal,
                                mu_spurious, low_rank_spurious, diag_spurious)
    mu = _sc_mu_gather(b)(mu_t, combo)
    cov = _assemble_cov(combo, cov_t, b)
    return (mu, cov)


# speedup vs baseline: 1.1721x; 1.0257x over previous
import functools

import jax
import jax.numpy as jnp
from jax import lax
from jax.experimental import pallas as pl
from jax.experimental.pallas import tpu as pltpu
from jax.experimental.pallas import tpu_sc as plsc

Z = 128
R = 64
NCOMBO = 4


def _softplus(x):
    return jnp.maximum(x, 0.0) + jnp.log1p(jnp.exp(-jnp.abs(x)))


def _table_kernel(mu_c_ref, lr_c_ref, d_c_ref, mu_s_ref, lr_s_ref, d_s_ref,
                  cov_t_ref, mu_t_ref):
    row = jax.lax.broadcasted_iota(jnp.int32, (Z, Z), 0)
    col = jax.lax.broadcasted_iota(jnp.int32, (Z, Z), 1)
    diag_mask = (row == col).astype(jnp.float32)
    zeros_blk = jnp.zeros((Z, Z), dtype=jnp.float32)
    for combo in range(NCOMBO):
        e = combo % 2
        lrc = lr_c_ref[e]
        cc = jax.lax.dot_general(lrc, lrc, (((1,), (1,)), ((), ())),
                                 preferred_element_type=jnp.float32)
        dc = _softplus(d_c_ref[e]) + 1e-6
        cc = cc + diag_mask * dc[None, :]
        lrs = lr_s_ref[combo]
        cs = jax.lax.dot_general(lrs, lrs, (((1,), (1,)), ((), ())),
                                 preferred_element_type=jnp.float32)
        ds = _softplus(d_s_ref[combo]) + 1e-6
        cs = cs + diag_mask * ds[None, :]
        cov_t_ref[combo, 0:Z, 0:Z] = cc
        cov_t_ref[combo, 0:Z, Z:2 * Z] = zeros_blk
        cov_t_ref[combo, Z:2 * Z, 0:Z] = zeros_blk
        cov_t_ref[combo, Z:2 * Z, Z:2 * Z] = cs
        mu_t_ref[combo, 0:Z] = mu_c_ref[e, :]
        mu_t_ref[combo, Z:2 * Z] = mu_s_ref[combo, :]


def _build_tables(mu_causal, low_rank_causal, diag_causal,
                  mu_spurious, low_rank_spurious, diag_spurious):
    mu_s = mu_spurious.reshape(NCOMBO, Z)
    lr_s = low_rank_spurious.reshape(NCOMBO, Z, R)
    d_s = diag_spurious.reshape(NCOMBO, Z)
    return pl.pallas_call(
        _table_kernel,
        out_shape=(
            jax.ShapeDtypeStruct((NCOMBO, 2 * Z, 2 * Z), jnp.float32),
            jax.ShapeDtypeStruct((NCOMBO, 2 * Z), jnp.float32),
        ),
    )(mu_causal, low_rank_causal, diag_causal, mu_s, lr_s, d_s)


W = 32


def _cov_kernel(combo_ref, cov_t_ref, cov_out_ref, sems):
    b = cov_out_ref.shape[0]

    def body(j, carry):
        @pl.when(j >= W)
        def _():
            pltpu.make_async_copy(cov_t_ref.at[0], cov_out_ref.at[j - W],
                                  sems.at[j % W]).wait()
        c = combo_ref[j]
        pltpu.make_async_copy(cov_t_ref.at[c], cov_out_ref.at[j],
                              sems.at[j % W]).start()
        return carry

    jax.lax.fori_loop(0, b, body, 0)

    def tail(j, carry):
        pltpu.make_async_copy(cov_t_ref.at[0], cov_out_ref.at[j],
                              sems.at[j % W]).wait()
        return carry

    jax.lax.fori_loop(b - W, b, tail, 0)


def _assemble_cov(combo, cov_t, b):
    return pl.pallas_call(
        _cov_kernel,
        grid_spec=pltpu.PrefetchScalarGridSpec(
            num_scalar_prefetch=1,
            grid=(1,),
            in_specs=[
                pl.BlockSpec((NCOMBO, 2 * Z, 2 * Z), lambda i, c: (0, 0, 0)),
            ],
            out_specs=pl.BlockSpec(memory_space=pl.ANY),
            scratch_shapes=[pltpu.SemaphoreType.DMA((W,))],
        ),
        out_shape=jax.ShapeDtypeStruct((b, 2 * Z, 2 * Z), jnp.float32),
    )(combo, cov_t)


def _sc_mu_gather(b):
    info = plsc.get_sparse_core_info()
    nc, ns = 1, info.num_subcores
    nw = nc * ns
    bpw = b // nw
    mesh = plsc.VectorSubcoreMesh(core_axis_name="c", subcore_axis_name="s",
                                  num_cores=nc)

    @functools.partial(
        pl.kernel,
        mesh=mesh,
        out_type=jax.ShapeDtypeStruct((b, 2 * Z), jnp.float32),
        scratch_types=[
            pltpu.VMEM((bpw,), jnp.int32),
            pltpu.VMEM((NCOMBO, 2 * Z), jnp.float32),
            pltpu.VMEM((bpw, 2 * Z), jnp.float32),
        ],
    )
    def k(mu_t, combo, mu_out, combo_v, mu_t_v, rows_v):
        wid = lax.axis_index("s") * nc + lax.axis_index("c")
        base = wid * bpw
        pltpu.sync_copy(combo.at[pl.ds(base, bpw)], combo_v)
        pltpu.sync_copy(mu_t, mu_t_v)
        cvecs = [combo_v[pl.ds(16 * m, 16)] for m in range(bpw // 16)]
        cs = [cvecs[m][i] for m in range(bpw // 16) for i in range(16)]
        for p in range(2 * Z // 16):
            t0 = mu_t_v[0, pl.ds(16 * p, 16)]
            t1 = mu_t_v[1, pl.ds(16 * p, 16)]
            t2 = mu_t_v[2, pl.ds(16 * p, 16)]
            t3 = mu_t_v[3, pl.ds(16 * p, 16)]
            for j in range(bpw):
                c = cs[j]
                r = jnp.where(c == 0, t0,
                              jnp.where(c == 1, t1,
                                        jnp.where(c == 2, t2, t3)))
                rows_v[j, pl.ds(16 * p, 16)] = r
        pltpu.sync_copy(rows_v, mu_out.at[pl.ds(base, bpw)])

    return k


def kernel(y, e, mu_causal, low_rank_causal, diag_causal,
           mu_spurious, low_rank_spurious, diag_spurious):
    combo = (y.astype(jnp.int32) * 2 + e.astype(jnp.int32))
    b = y.shape[0]
    cov_t, mu_t = _build_tables(mu_causal, low_rank_causal, diag_causal,
                                mu_spurious, low_rank_spurious, diag_spurious)
    mu = _sc_mu_gather(b)(mu_t, combo)
    cov = _assemble_cov(combo, cov_t, b)
    return (mu, cov)


# speedup vs baseline: 1.1879x; 1.0135x over previous
import functools

import jax
import jax.numpy as jnp
from jax import lax
from jax.experimental import pallas as pl
from jax.experimental.pallas import tpu as pltpu
from jax.experimental.pallas import tpu_sc as plsc

Z = 128
R = 64
NCOMBO = 4


def _softplus(x):
    return jnp.maximum(x, 0.0) + jnp.log1p(jnp.exp(-jnp.abs(x)))


def _mu_table_kernel(mu_c_ref, mu_s_ref, mu_t_ref):
    for combo in range(NCOMBO):
        e = combo % 2
        mu_t_ref[combo, 0:Z] = mu_c_ref[e, :]
        mu_t_ref[combo, Z:2 * Z] = mu_s_ref[combo, :]


def _build_mu_table(mu_causal, mu_spurious):
    return pl.pallas_call(
        _mu_table_kernel,
        out_shape=jax.ShapeDtypeStruct((NCOMBO, 2 * Z), jnp.float32),
    )(mu_causal, mu_spurious.reshape(NCOMBO, Z))


W = 32


def _cov_kernel(combo_ref, lr_c_ref, d_c_ref, lr_s_ref, d_s_ref,
                cov_out_ref, cov_t_ref, sems):
    b = cov_out_ref.shape[0]
    row = jax.lax.broadcasted_iota(jnp.int32, (Z, Z), 0)
    col = jax.lax.broadcasted_iota(jnp.int32, (Z, Z), 1)
    diag_mask = (row == col).astype(jnp.float32)
    zeros_blk = jnp.zeros((Z, Z), dtype=jnp.float32)
    for combo in range(NCOMBO):
        e = combo % 2
        lrc = lr_c_ref[e]
        cc = jax.lax.dot_general(lrc, lrc, (((1,), (1,)), ((), ())),
                                 preferred_element_type=jnp.float32)
        dc = _softplus(d_c_ref[e]) + 1e-6
        cc = cc + diag_mask * dc[None, :]
        lrs = lr_s_ref[combo]
        cs = jax.lax.dot_general(lrs, lrs, (((1,), (1,)), ((), ())),
                                 preferred_element_type=jnp.float32)
        ds = _softplus(d_s_ref[combo]) + 1e-6
        cs = cs + diag_mask * ds[None, :]
        cov_t_ref[combo, 0:Z, 0:Z] = cc
        cov_t_ref[combo, 0:Z, Z:2 * Z] = zeros_blk
        cov_t_ref[combo, Z:2 * Z, 0:Z] = zeros_blk
        cov_t_ref[combo, Z:2 * Z, Z:2 * Z] = cs

    def body(j, carry):
        @pl.when(j >= W)
        def _():
            pltpu.make_async_copy(cov_t_ref.at[0], cov_out_ref.at[j - W],
                                  sems.at[j % W]).wait()
        c = combo_ref[j]
        pltpu.make_async_copy(cov_t_ref.at[c], cov_out_ref.at[j],
                              sems.at[j % W]).start()
        return carry

    jax.lax.fori_loop(0, b, body, 0)

    def tail(j, carry):
        pltpu.make_async_copy(cov_t_ref.at[0], cov_out_ref.at[j],
                              sems.at[j % W]).wait()
        return carry

    jax.lax.fori_loop(b - W, b, tail, 0)


def _assemble_cov(combo, low_rank_causal, diag_causal,
                  low_rank_spurious, diag_spurious, b):
    lr_s = low_rank_spurious.reshape(NCOMBO, Z, R)
    d_s = diag_spurious.reshape(NCOMBO, Z)
    return pl.pallas_call(
        _cov_kernel,
        grid_spec=pltpu.PrefetchScalarGridSpec(
            num_scalar_prefetch=1,
            grid=(1,),
            in_specs=[
                pl.BlockSpec((2, Z, R), lambda i, c: (0, 0, 0)),
                pl.BlockSpec((2, Z), lambda i, c: (0, 0)),
                pl.BlockSpec((NCOMBO, Z, R), lambda i, c: (0, 0, 0)),
                pl.BlockSpec((NCOMBO, Z), lambda i, c: (0, 0)),
            ],
            out_specs=pl.BlockSpec(memory_space=pl.ANY),
            scratch_shapes=[
                pltpu.VMEM((NCOMBO, 2 * Z, 2 * Z), jnp.float32),
                pltpu.SemaphoreType.DMA((W,)),
            ],
        ),
        out_shape=jax.ShapeDtypeStruct((b, 2 * Z, 2 * Z), jnp.float32),
    )(combo, low_rank_causal, diag_causal, lr_s, d_s)


def _sc_mu_gather(b):
    info = plsc.get_sparse_core_info()
    nc, ns = 1, info.num_subcores
    nw = nc * ns
    bpw = b // nw
    mesh = plsc.VectorSubcoreMesh(core_axis_name="c", subcore_axis_name="s",
                                  num_cores=nc)

    @functools.partial(
        pl.kernel,
        mesh=mesh,
        out_type=jax.ShapeDtypeStruct((b, 2 * Z), jnp.float32),
        scratch_types=[
            pltpu.VMEM((bpw,), jnp.int32),
            pltpu.VMEM((NCOMBO, 2 * Z), jnp.float32),
            pltpu.VMEM((bpw, 2 * Z), jnp.float32),
        ],
    )
    def k(mu_t, combo, mu_out, combo_v, mu_t_v, rows_v):
        wid = lax.axis_index("s") * nc + lax.axis_index("c")
        base = wid * bpw
        pltpu.sync_copy(combo.at[pl.ds(base, bpw)], combo_v)
        pltpu.sync_copy(mu_t, mu_t_v)
        cvecs = [combo_v[pl.ds(16 * m, 16)] for m in range(bpw // 16)]
        cs = [cvecs[m][i] for m in range(bpw // 16) for i in range(16)]
        for p in range(2 * Z // 16):
            t0 = mu_t_v[0, pl.ds(16 * p, 16)]
            t1 = mu_t_v[1, pl.ds(16 * p, 16)]
            t2 = mu_t_v[2, pl.ds(16 * p, 16)]
            t3 = mu_t_v[3, pl.ds(16 * p, 16)]
            for j in range(bpw):
                c = cs[j]
                r = jnp.where(c == 0, t0,
                              jnp.where(c == 1, t1,
                                        jnp.where(c == 2, t2, t3)))
                rows_v[j, pl.ds(16 * p, 16)] = r
        pltpu.sync_copy(rows_v, mu_out.at[pl.ds(base, bpw)])

    return k


def kernel(y, e, mu_causal, low_rank_causal, diag_causal,
           mu_spurious, low_rank_spurious, diag_spurious):
    combo = (y.astype(jnp.int32) * 2 + e.astype(jnp.int32))
    b = y.shape[0]
    mu_t = _build_mu_table(mu_causal, mu_spurious)
    mu = _sc_mu_gather(b)(mu_t, combo)
    cov = _assemble_cov(combo, low_rank_causal, diag_causal,
                        low_rank_spurious, diag_spurious, b)
    return (mu, cov)
